# Initial kernel scaffold; baseline (speedup 1.0000x reference)
import jax
import jax.numpy as jnp
from jax.experimental import pallas as pl


def kernel(x, gate_w, w1, b1, w2, b2):
    raise NotImplementedError("write your pallas kernel here")


# speedup vs baseline: 4.0509x; 4.0509x over previous
import functools

import jax
import jax.numpy as jnp
from jax import lax
from jax.experimental import pallas as pl
from jax.experimental.pallas import tpu as pltpu
from jax.experimental.pallas import tpu_sc as plsc

_E = 8
_K = 2
_D = 1024
_F = 4096
_N = 4096
_BM = 256
_NBLK = 40
_PAD = _BM * _NBLK
_TB = 512

_NC = 2
_NS = 16
_NW = _NC * _NS
_ROWS_W = _PAD // _NW
_GCH = 40
_GNCH = _ROWS_W // _GCH
_TOK_W = _N // _NW
_CT = 16
_CNCH = _TOK_W // _CT


def _router_body(x_ref, gw_ref, w_ref, i_ref, ps_ref):
    step = pl.program_id(0)
    x = x_ref[...]
    gw = gw_ref[...]
    logits = lax.dot_general(x, gw, (((1,), (1,)), ((), ())),
                             preferred_element_type=jnp.float32)
    m = jnp.max(logits, axis=1, keepdims=True)
    ex = jnp.exp(logits - m)
    probs = ex / jnp.sum(ex, axis=1, keepdims=True)

    @pl.when(step == 0)
    def _():
        ps_ref[...] = jnp.zeros_like(ps_ref)

    ps_ref[...] += jnp.sum(probs, axis=0, keepdims=True)

    col = lax.broadcasted_iota(jnp.int32, (_TB, _E), 1)
    m1 = jnp.max(probs, axis=1, keepdims=True)
    i1 = jnp.min(jnp.where(probs == m1, col, _E), axis=1, keepdims=True)
    pm = jnp.where(col == i1, -1.0, probs)
    m2 = jnp.max(pm, axis=1, keepdims=True)
    i2 = jnp.min(jnp.where(pm == m2, col, _E), axis=1, keepdims=True)
    s = m1 + m2 + 1e-9
    w_ref[...] = jnp.where(col == 0, m1 / s, jnp.where(col == 1, m2 / s, 0.0))
    i_ref[...] = jnp.where(col == 0, i1, jnp.where(col == 1, i2, 0))


def _router(x2d, gate_w):
    return pl.pallas_call(
        _router_body,
        grid=(_N // _TB,),
        in_specs=[
            pl.BlockSpec((_TB, _D), lambda i: (i, 0)),
            pl.BlockSpec((_E, _D), lambda i: (0, 0)),
        ],
        out_specs=[
            pl.BlockSpec((_TB, _E), lambda i: (i, 0)),
            pl.BlockSpec((_TB, _E), lambda i: (i, 0)),
            pl.BlockSpec((1, _E), lambda i: (0, 0)),
        ],
        out_shape=[
            jax.ShapeDtypeStruct((_N, _E), jnp.float32),
            jax.ShapeDtypeStruct((_N, _E), jnp.int32),
            jax.ShapeDtypeStruct((1, _E), jnp.float32),
        ],
    )(x2d, gate_w)


def _ffn_body(be_ref, x_ref, w1_ref, b1_ref, w2_ref, b2_ref, ws_ref, out_ref):
    x = x_ref[...].astype(jnp.bfloat16)
    h = lax.dot_general(x, w1_ref[0], (((1,), (1,)), ((), ())),
                        preferred_element_type=jnp.float32)
    h = h + b1_ref[0]
    h = 0.5 * h * (1.0 + lax.erf(h * (2.0 ** -0.5)))
    y = lax.dot_general(h.astype(jnp.bfloat16), w2_ref[0],
                        (((1,), (1,)), ((), ())),
                        preferred_element_type=jnp.float32)
    y = y + b2_ref[0]
    out_ref[...] = y * ws_ref[...]


def _ffn(x_sorted, w1, b1, w2, b2, w_sorted, block_expert):
    grid_spec = pltpu.PrefetchScalarGridSpec(
        num_scalar_prefetch=1,
        grid=(_NBLK,),
        in_specs=[
            pl.BlockSpec((_BM, _D), lambda i, be: (i, 0)),
            pl.BlockSpec((1, _F, _D), lambda i, be: (be[i], 0, 0)),
            pl.BlockSpec((1, 1, _F), lambda i, be: (be[i], 0, 0)),
            pl.BlockSpec((1, _D, _F), lambda i, be: (be[i], 0, 0)),
            pl.BlockSpec((1, 1, _D), lambda i, be: (be[i], 0, 0)),
            pl.BlockSpec((_BM, 1), lambda i, be: (i, 0)),
        ],
        out_specs=pl.BlockSpec((_BM, _D), lambda i, be: (i, 0)),
    )
    return pl.pallas_call(
        _ffn_body,
        grid_spec=grid_spec,
        out_shape=jax.ShapeDtypeStruct((_PAD, _D), jnp.float32),
    )(block_expert, x_sorted, w1.astype(jnp.bfloat16),
      b1.reshape(_E, 1, _F), w2.astype(jnp.bfloat16),
      b2.reshape(_E, 1, _D), w_sorted)


def _sc_mesh():
    return plsc.VectorSubcoreMesh(core_axis_name="c", subcore_axis_name="s",
                                  num_cores=_NC, num_subcores=_NS)


def _sc_gather_call(x2d, gidx):
    @functools.partial(
        pl.kernel,
        out_type=jax.ShapeDtypeStruct((_PAD, _D), jnp.float32),
        mesh=_sc_mesh(),
        scratch_types=[
            pltpu.VMEM((_GNCH, _GCH), jnp.int32),
            pltpu.VMEM((2, _GCH, _D), jnp.float32),
            pltpu.SemaphoreType.DMA,
            pltpu.SemaphoreType.DMA,
        ],
    )
    def k(x_hbm, idx_hbm, out_hbm, idx_v, rows_v, gsem, ssem):
        wid = lax.axis_index("s") * _NC + lax.axis_index("c")
        base = wid * _ROWS_W
        pltpu.sync_copy(idx_hbm.at[wid], idx_v)
        g = [None] * _GNCH
        st = [None] * _GNCH
        g[0] = pltpu.async_copy(x_hbm.at[idx_v.at[0]], rows_v.at[0], gsem)
        for ch in range(_GNCH):
            b = ch & 1
            if ch + 1 < _GNCH:
                if ch >= 1:
                    st[ch - 1].wait()
                g[ch + 1] = pltpu.async_copy(
                    x_hbm.at[idx_v.at[ch + 1]], rows_v.at[(ch + 1) & 1], gsem)
            g[ch].wait()
            st[ch] = pltpu.async_copy(
                rows_v.at[b], out_hbm.at[pl.ds(base + ch * _GCH, _GCH)], ssem)
        st[_GNCH - 2].wait()
        st[_GNCH - 1].wait()

    return k(x2d, gidx)


def _sc_combine_call(y_sorted, cidx):
    @functools.partial(
        pl.kernel,
        out_type=jax.ShapeDtypeStruct((_N, _D), jnp.float32),
        mesh=_sc_mesh(),
        scratch_types=[
            pltpu.VMEM((_CNCH, 2 * _CT), jnp.int32),
            pltpu.VMEM((2, 2 * _CT, _D), jnp.float32),
            pltpu.VMEM((2, _CT, _D), jnp.float32),
            pltpu.SemaphoreType.DMA,
            pltpu.SemaphoreType.DMA,
        ],
    )
    def k(y_hbm, idx_hbm, out_hbm, idx_v, rows_v, out_v, gsem, ssem):
        wid = lax.axis_index("s") * _NC + lax.axis_index("c")
        base = wid * _TOK_W
        pltpu.sync_copy(idx_hbm.at[wid], idx_v)
        g = [None] * _CNCH
        st = [None] * _CNCH
        g[0] = pltpu.async_copy(y_hbm.at[idx_v.at[0]], rows_v.at[0], gsem)
        for ch in range(_CNCH):
            b = ch & 1
            if ch + 1 < _CNCH:
                if ch >= 1:
                    st[ch - 1].wait()
                g[ch + 1] = pltpu.async_copy(
                    y_hbm.at[idx_v.at[ch + 1]], rows_v.at[(ch + 1) & 1], gsem)
            g[ch].wait()

            def tok_body(i, _, b=b):
                for l in range(_D // 16):
                    sl = pl.ds(l * 16, 16)
                    out_v[b, i, sl] = (rows_v[b, 2 * i, sl]
                                       + rows_v[b, 2 * i + 1, sl])
                return 0

            lax.fori_loop(0, _CT, tok_body, 0)
            st[ch] = pltpu.async_copy(
                out_v.at[b], out_hbm.at[pl.ds(base + ch * _CT, _CT)], ssem)
        st[_CNCH - 2].wait()
        st[_CNCH - 1].wait()

    return k(y_sorted, cidx)


def kernel(x, gate_w, w1, b1, w2, b2):
    B, T, D = x.shape
    x2d = x.reshape(_N, _D)
    wts, idx, psum = _router(x2d, gate_w)

    avg = psum[0] / _N
    aux = jnp.mean((avg - 1.0 / _E) ** 2) * _E

    e_flat = idx[:, :_K].reshape(-1)
    w_flat = wts[:, :_K].reshape(-1)
    onehot = (e_flat[:, None] == jnp.arange(_E, dtype=jnp.int32)[None, :]
              ).astype(jnp.int32)
    counts = jnp.sum(onehot, axis=0)
    rank = jnp.cumsum(onehot, axis=0) - onehot
    rank = jnp.take_along_axis(rank, e_flat[:, None], axis=1)[:, 0]
    blocks_e = (counts + _BM - 1) // _BM
    cum_blocks = jnp.cumsum(blocks_e)
    bstart = cum_blocks - blocks_e
    pos = bstart[e_flat] * _BM + rank
    ar = jnp.arange(_N * _K, dtype=jnp.int32)
    gidx = jnp.zeros((_PAD,), jnp.int32).at[pos].set(ar // _K)
    wsort = jnp.zeros((_PAD,), jnp.float32).at[pos].set(w_flat)
    block_expert = jnp.minimum(
        jnp.searchsorted(cum_blocks, jnp.arange(_NBLK), side="right"), _E - 1
    ).astype(jnp.int32)

    x_sorted = _sc_gather_call(x2d, gidx.reshape(_NW, _GNCH, _GCH))
    y_sorted = _ffn(x_sorted, w1, b1, w2, b2,
                    wsort.reshape(_PAD, 1), block_expert)
    out2d = _sc_combine_call(
        y_sorted, pos.reshape(_NW, _CNCH, 2 * _CT).astype(jnp.int32))
    return out2d.reshape(B, T, D), aux


# speedup vs baseline: 4.8466x; 1.1964x over previous
import functools

import jax
import jax.numpy as jnp
from jax import lax
from jax.experimental import pallas as pl
from jax.experimental.pallas import tpu as pltpu
from jax.experimental.pallas import tpu_sc as plsc

_E = 8
_K = 2
_D = 1024
_F = 4096
_N = 4096
_BM = 256
_NBLK = 40
_PAD = _BM * _NBLK
_TB = 512

_NC = 2
_NS = 16
_NW = _NC * _NS
_ROWS_W = _PAD // _NW
_GCH = 40
_GNCH = _ROWS_W // _GCH
_TOK_W = _N // _NW
_CT = 16
_CNCH = _TOK_W // _CT


def _router_body(x_ref, gw_ref, w_ref, i_ref, ps_ref):
    step = pl.program_id(0)
    x = x_ref[...]
    gw = gw_ref[...]
    logits = lax.dot_general(x, gw, (((1,), (1,)), ((), ())),
                             preferred_element_type=jnp.float32)
    m = jnp.max(logits, axis=1, keepdims=True)
    ex = jnp.exp(logits - m)
    probs = ex / jnp.sum(ex, axis=1, keepdims=True)

    @pl.when(step == 0)
    def _():
        ps_ref[...] = jnp.zeros_like(ps_ref)

    ps_ref[...] += jnp.sum(probs, axis=0, keepdims=True)

    col = lax.broadcasted_iota(jnp.int32, (_TB, _E), 1)
    m1 = jnp.max(probs, axis=1, keepdims=True)
    i1 = jnp.min(jnp.where(probs == m1, col, _E), axis=1, keepdims=True)
    pm = jnp.where(col == i1, -1.0, probs)
    m2 = jnp.max(pm, axis=1, keepdims=True)
    i2 = jnp.min(jnp.where(pm == m2, col, _E), axis=1, keepdims=True)
    s = m1 + m2 + 1e-9
    w_ref[...] = jnp.where(col == 0, m1 / s, jnp.where(col == 1, m2 / s, 0.0))
    i_ref[...] = jnp.where(col == 0, i1, jnp.where(col == 1, i2, 0))


def _router(x2d, gate_w):
    return pl.pallas_call(
        _router_body,
        grid=(_N // _TB,),
        in_specs=[
            pl.BlockSpec((_TB, _D), lambda i: (i, 0)),
            pl.BlockSpec((_E, _D), lambda i: (0, 0)),
        ],
        out_specs=[
            pl.BlockSpec((_TB, _E), lambda i: (i, 0)),
            pl.BlockSpec((_TB, _E), lambda i: (i, 0)),
            pl.BlockSpec((1, _E), lambda i: (0, 0)),
        ],
        out_shape=[
            jax.ShapeDtypeStruct((_N, _E), jnp.float32),
            jax.ShapeDtypeStruct((_N, _E), jnp.int32),
            jax.ShapeDtypeStruct((1, _E), jnp.float32),
        ],
    )(x2d, gate_w)


def _ffn_body(be_ref, x_ref, w1_ref, b1_ref, w2_ref, b2_ref, ws_ref, out_ref):
    x = x_ref[...].astype(jnp.bfloat16)
    h = lax.dot_general(x, w1_ref[0], (((1,), (1,)), ((), ())),
                        preferred_element_type=jnp.float32)
    h = h + b1_ref[0]
    h = 0.5 * h * (1.0 + lax.erf(h * (2.0 ** -0.5)))
    y = lax.dot_general(h.astype(jnp.bfloat16), w2_ref[0],
                        (((1,), (1,)), ((), ())),
                        preferred_element_type=jnp.float32)
    y = y + b2_ref[0]
    out_ref[...] = y * ws_ref[...]


def _ffn(x_sorted, w1, b1, w2, b2, w_sorted, block_expert):
    grid_spec = pltpu.PrefetchScalarGridSpec(
        num_scalar_prefetch=1,
        grid=(_NBLK,),
        in_specs=[
            pl.BlockSpec((_BM, _D), lambda i, be: (i, 0)),
            pl.BlockSpec((1, _F, _D), lambda i, be: (be[i], 0, 0)),
            pl.BlockSpec((1, 1, _F), lambda i, be: (be[i], 0, 0)),
            pl.BlockSpec((1, _D, _F), lambda i, be: (be[i], 0, 0)),
            pl.BlockSpec((1, 1, _D), lambda i, be: (be[i], 0, 0)),
            pl.BlockSpec((_BM, 1), lambda i, be: (i, 0)),
        ],
        out_specs=pl.BlockSpec((_BM, _D), lambda i, be: (i, 0)),
    )
    return pl.pallas_call(
        _ffn_body,
        grid_spec=grid_spec,
        out_shape=jax.ShapeDtypeStruct((_PAD, _D), jnp.float32),
    )(block_expert, x_sorted, w1.astype(jnp.bfloat16),
      b1.reshape(_E, 1, _F), w2.astype(jnp.bfloat16),
      b2.reshape(_E, 1, _D), w_sorted)


def _sc_mesh():
    return plsc.VectorSubcoreMesh(core_axis_name="c", subcore_axis_name="s",
                                  num_cores=_NC, num_subcores=_NS)


def _sc_gather_call(x2d, gidx):
    @functools.partial(
        pl.kernel,
        out_type=jax.ShapeDtypeStruct((_PAD, _D), jnp.float32),
        mesh=_sc_mesh(),
        scratch_types=[
            pltpu.VMEM((_GNCH, _GCH), jnp.int32),
            pltpu.VMEM((3, _GCH, _D), jnp.float32),
            pltpu.SemaphoreType.DMA,
            pltpu.SemaphoreType.DMA,
        ],
    )
    def k(x_hbm, idx_hbm, out_hbm, idx_v, rows_v, gsem, ssem):
        wid = lax.axis_index("s") * _NC + lax.axis_index("c")
        base = wid * _ROWS_W
        pltpu.sync_copy(idx_hbm.at[wid], idx_v)
        nb = 3
        g = [None] * _GNCH
        st = [None] * _GNCH
        for ch in range(nb - 1):
            g[ch] = pltpu.async_copy(
                x_hbm.at[idx_v.at[ch]], rows_v.at[ch % nb], gsem)
        for ch in range(_GNCH):
            b = ch % nb
            g[ch].wait()
            st[ch] = pltpu.async_copy(
                rows_v.at[b], out_hbm.at[pl.ds(base + ch * _GCH, _GCH)], ssem)
            nxt = ch + nb - 1
            if nxt < _GNCH:
                if nxt - nb >= 0:
                    st[nxt - nb].wait()
                g[nxt] = pltpu.async_copy(
                    x_hbm.at[idx_v.at[nxt]], rows_v.at[nxt % nb], gsem)
        for ch in range(max(0, _GNCH - nb), _GNCH):
            if st[ch] is not None:
                st[ch].wait()

    return k(x2d, gidx)


def _sc_combine_call(y_sorted, cidx):
    @functools.partial(
        pl.kernel,
        out_type=jax.ShapeDtypeStruct((_N, _D), jnp.float32),
        mesh=_sc_mesh(),
        scratch_types=[
            pltpu.VMEM((_CNCH, 2 * _CT), jnp.int32),
            pltpu.VMEM((2, 2 * _CT, _D), jnp.float32),
            pltpu.VMEM((2, _CT, _D), jnp.float32),
            pltpu.SemaphoreType.DMA,
            pltpu.SemaphoreType.DMA,
        ],
    )
    def k(y_hbm, idx_hbm, out_hbm, idx_v, rows_v, out_v, gsem, ssem):
        wid = lax.axis_index("s") * _NC + lax.axis_index("c")
        base = wid * _TOK_W
        pltpu.sync_copy(idx_hbm.at[wid], idx_v)
        g = [None] * _CNCH
        st = [None] * _CNCH
        g[0] = pltpu.async_copy(y_hbm.at[idx_v.at[0]], rows_v.at[0], gsem)
        for ch in range(_CNCH):
            b = ch & 1
            if ch + 1 < _CNCH:
                if ch >= 1:
                    st[ch - 1].wait()
                g[ch + 1] = pltpu.async_copy(
                    y_hbm.at[idx_v.at[ch + 1]], rows_v.at[(ch + 1) & 1], gsem)
            g[ch].wait()

            def tok_body(i, _, b=b):
                for l in range(_D // 16):
                    sl = pl.ds(l * 16, 16)
                    out_v[b, i, sl] = (rows_v[b, 2 * i, sl]
                                       + rows_v[b, 2 * i + 1, sl])
                return 0

            lax.fori_loop(0, _CT, tok_body, 0)
            st[ch] = pltpu.async_copy(
                out_v.at[b], out_hbm.at[pl.ds(base + ch * _CT, _CT)], ssem)
        st[_CNCH - 2].wait()
        st[_CNCH - 1].wait()

    return k(y_sorted, cidx)


def kernel(x, gate_w, w1, b1, w2, b2):
    B, T, D = x.shape
    x2d = x.reshape(_N, _D)
    wts, idx, psum = _router(x2d, gate_w)

    avg = psum[0] / _N
    aux = jnp.mean((avg - 1.0 / _E) ** 2) * _E

    e_flat = idx[:, :_K].reshape(-1)
    w_flat = wts[:, :_K].reshape(-1)
    onehot = (e_flat[:, None] == jnp.arange(_E, dtype=jnp.int32)[None, :]
              ).astype(jnp.int32)
    counts = jnp.sum(onehot, axis=0)
    rank = jnp.cumsum(onehot, axis=0) - onehot
    rank = jnp.take_along_axis(rank, e_flat[:, None], axis=1)[:, 0]
    blocks_e = (counts + _BM - 1) // _BM
    cum_blocks = jnp.cumsum(blocks_e)
    bstart = cum_blocks - blocks_e
    pos = bstart[e_flat] * _BM + rank
    ar = jnp.arange(_N * _K, dtype=jnp.int32)
    gidx = (jnp.arange(_PAD, dtype=jnp.int32) % _N).at[pos].set(ar // _K)
    wsort = jnp.zeros((_PAD,), jnp.float32).at[pos].set(w_flat)
    block_expert = jnp.minimum(
        jnp.searchsorted(cum_blocks, jnp.arange(_NBLK), side="right"), _E - 1
    ).astype(jnp.int32)

    x_sorted = _sc_gather_call(x2d, gidx.reshape(_NW, _GNCH, _GCH))
    y_sorted = _ffn(x_sorted, w1, b1, w2, b2,
                    wsort.reshape(_PAD, 1), block_expert)
    out2d = _sc_combine_call(
        y_sorted, pos.reshape(_NW, _CNCH, 2 * _CT).astype(jnp.int32))
    return out2d.reshape(B, T, D), aux


# speedup vs baseline: 5.7964x; 1.1960x over previous
import functools

import jax
import jax.numpy as jnp
from jax import lax
from jax.experimental import pallas as pl
from jax.experimental.pallas import tpu as pltpu
from jax.experimental.pallas import tpu_sc as plsc

_E = 8
_K = 2
_D = 1024
_F = 4096
_N = 4096
_BM = 256
_NBLK = 40
_PAD = _BM * _NBLK
_TB = 512
_NTB = _N // _TB

_NC = 2
_NS = 16
_NW = _NC * _NS
_TOK_W = _N // _NW
_CT = 32
_DNCH = _TOK_W // _CT
_CCT = 16
_CNCH = _TOK_W // _CCT


def _top2(probs, col):
    m1 = jnp.max(probs, axis=1, keepdims=True)
    i1 = jnp.min(jnp.where(probs == m1, col, _E), axis=1, keepdims=True)
    pm = jnp.where(col == i1, -1.0, probs)
    m2 = jnp.max(pm, axis=1, keepdims=True)
    i2 = jnp.min(jnp.where(pm == m2, col, _E), axis=1, keepdims=True)
    return m1, i1, m2, i2


def _router_body(x_ref, gw_ref, w_ref, pos_ref, ps_ref, be_ref, cnt_ref):
    p = pl.program_id(0)
    blk = pl.program_id(1)
    x = x_ref[...]
    gw = gw_ref[...]
    logits = lax.dot_general(x, gw, (((1,), (1,)), ((), ())),
                             preferred_element_type=jnp.float32)
    m = jnp.max(logits, axis=1, keepdims=True)
    ex = jnp.exp(logits - m)
    probs = ex / jnp.sum(ex, axis=1, keepdims=True)
    col = lax.broadcasted_iota(jnp.int32, (_TB, _E), 1)
    m1, i1, m2, i2 = _top2(probs, col)
    oh1 = jnp.where(col == i1, 1.0, 0.0)
    oh2 = jnp.where(col == i2, 1.0, 0.0)

    @pl.when(p == 0)
    def _pass1():
        @pl.when(blk == 0)
        def _():
            ps_ref[...] = jnp.zeros_like(ps_ref)

        ps_ref[...] += jnp.sum(probs, axis=0, keepdims=True)
        cnt_ref[blk, :] = jnp.sum(oh1 + oh2, axis=0)

    @pl.when(p == 1)
    def _pass2():
        s = m1 + m2 + 1e-9
        col32 = lax.broadcasted_iota(jnp.int32, (_TB, 2 * 16), 1)
        w_ref[...] = jnp.where(col32 < 16, m1 / s, m2 / s)
        cnt = cnt_ref[...]
        totals = jnp.sum(cnt, axis=0, keepdims=True)
        blocks_e = jnp.floor((totals + (_BM - 1)) * (1.0 / _BM))
        ecol = lax.broadcasted_iota(jnp.int32, (_E, _E), 1)
        erow = lax.broadcasted_iota(jnp.int32, (_E, _E), 0)
        tri_inc = jnp.where(erow <= ecol, 1.0, 0.0)
        cumb = lax.dot_general(blocks_e, tri_inc,
                               (((1,), (0,)), ((), ())),
                               preferred_element_type=jnp.float32)
        bstart = cumb - blocks_e
        brow = lax.broadcasted_iota(jnp.int32, (_NTB, _E), 0)
        bprefix = jnp.sum(jnp.where(brow < blk, cnt, 0.0), axis=0,
                          keepdims=True)
        trow = lax.broadcasted_iota(jnp.int32, (_TB, _TB), 0)
        tcol = lax.broadcasted_iota(jnp.int32, (_TB, _TB), 1)
        ltri = jnp.where(tcol < trow, 1.0, 0.0)
        pfx = lax.dot_general(ltri, oh1 + oh2, (((1,), (0,)), ((), ())),
                              preferred_element_type=jnp.float32)
        base = bstart * _BM + bprefix + pfx
        pos0 = jnp.sum(oh1 * base, axis=1, keepdims=True)
        pos1 = jnp.sum(oh2 * base, axis=1, keepdims=True)
        pos_ref[...] = jnp.where(col == 0, pos0,
                                 jnp.where(col == 1, pos1, 0.0)
                                 ).astype(jnp.int32)

        @pl.when(blk == 0)
        def _():
            bro = lax.broadcasted_iota(jnp.int32, (_NBLK, _E), 0
                                       ).astype(jnp.float32)
            be = jnp.sum(jnp.where(cumb <= bro, 1.0, 0.0), axis=1,
                         keepdims=True)
            be_ref[...] = jnp.minimum(be, _E - 1).astype(jnp.int32)


def _router(x2d, gate_w):
    return pl.pallas_call(
        _router_body,
        grid=(2, _NTB),
        in_specs=[
            pl.BlockSpec((_TB, _D), lambda p, b: (b, 0)),
            pl.BlockSpec((_E, _D), lambda p, b: (0, 0)),
        ],
        out_specs=[
            pl.BlockSpec((_TB, 32), lambda p, b: (b, 0)),
            pl.BlockSpec((_TB, _E), lambda p, b: (b, 0)),
            pl.BlockSpec((1, _E), lambda p, b: (0, 0)),
            pl.BlockSpec((_NBLK, 1), lambda p, b: (0, 0)),
        ],
        out_shape=[
            jax.ShapeDtypeStruct((_N, 32), jnp.float32),
            jax.ShapeDtypeStruct((_N, _E), jnp.int32),
            jax.ShapeDtypeStruct((1, _E), jnp.float32),
            jax.ShapeDtypeStruct((_NBLK, 1), jnp.int32),
        ],
        scratch_shapes=[pltpu.VMEM((_NTB, _E), jnp.float32)],
    )(x2d, gate_w)


def _ffn_body(be_ref, x_ref, w1_ref, b1_ref, w2_ref, b2_ref, out_ref):
    x = x_ref[...].astype(jnp.bfloat16)
    h = lax.dot_general(x, w1_ref[0], (((1,), (1,)), ((), ())),
                        preferred_element_type=jnp.float32)
    h = h + b1_ref[0]
    h = 0.5 * h * (1.0 + lax.erf(h * (2.0 ** -0.5)))
    y = lax.dot_general(h.astype(jnp.bfloat16), w2_ref[0],
                        (((1,), (1,)), ((), ())),
                        preferred_element_type=jnp.float32)
    out_ref[...] = y + b2_ref[0]


def _ffn(x_sorted, w1, b1, w2, b2, block_expert):
    grid_spec = pltpu.PrefetchScalarGridSpec(
        num_scalar_prefetch=1,
        grid=(_NBLK,),
        in_specs=[
            pl.BlockSpec((_BM, _D), lambda i, be: (i, 0)),
            pl.BlockSpec((1, _F, _D), lambda i, be: (be[i], 0, 0)),
            pl.BlockSpec((1, 1, _F), lambda i, be: (be[i], 0, 0)),
            pl.BlockSpec((1, _D, _F), lambda i, be: (be[i], 0, 0)),
            pl.BlockSpec((1, 1, _D), lambda i, be: (be[i], 0, 0)),
        ],
        out_specs=pl.BlockSpec((_BM, _D), lambda i, be: (i, 0)),
    )
    return pl.pallas_call(
        _ffn_body,
        grid_spec=grid_spec,
        out_shape=jax.ShapeDtypeStruct((_PAD, _D), jnp.float32),
    )(block_expert, x_sorted, w1.astype(jnp.bfloat16),
      b1.reshape(_E, 1, _F), w2.astype(jnp.bfloat16),
      b2.reshape(_E, 1, _D))


def _sc_mesh():
    return plsc.VectorSubcoreMesh(core_axis_name="c", subcore_axis_name="s",
                                  num_cores=_NC, num_subcores=_NS)


def _sc_dispatch_call(x2d, pidx):
    @functools.partial(
        pl.kernel,
        out_type=jax.ShapeDtypeStruct((_PAD, _D), jnp.float32),
        mesh=_sc_mesh(),
        scratch_types=[
            pltpu.VMEM((2 * _DNCH, _CT), jnp.int32),
            pltpu.VMEM((2, _CT, _D), jnp.float32),
            pltpu.SemaphoreType.DMA,
            pltpu.SemaphoreType.DMA,
        ],
    )
    def k(x_hbm, idx_hbm, out_hbm, idx_v, rows_v, lsem, ssem):
        wid = lax.axis_index("s") * _NC + lax.axis_index("c")
        tbase = wid * _TOK_W
        pltpu.sync_copy(idx_hbm.at[wid], idx_v)
        ld = [None] * _DNCH
        st = [[None, None] for _ in range(_DNCH)]
        ld[0] = pltpu.async_copy(
            x_hbm.at[pl.ds(tbase, _CT)], rows_v.at[0], lsem)
        for ch in range(_DNCH):
            b = ch & 1
            if ch + 1 < _DNCH:
                if ch >= 1:
                    st[ch - 1][0].wait()
                    st[ch - 1][1].wait()
                ld[ch + 1] = pltpu.async_copy(
                    x_hbm.at[pl.ds(tbase + (ch + 1) * _CT, _CT)],
                    rows_v.at[(ch + 1) & 1], lsem)
            ld[ch].wait()
            st[ch][0] = pltpu.async_copy(
                rows_v.at[b], out_hbm.at[idx_v.at[2 * ch]], ssem)
            st[ch][1] = pltpu.async_copy(
                rows_v.at[b], out_hbm.at[idx_v.at[2 * ch + 1]], ssem)
        for ch in (_DNCH - 2, _DNCH - 1):
            st[ch][0].wait()
            st[ch][1].wait()

    return k(x2d, pidx)


def _sc_combine_call(y_sorted, cidx, cw):
    @functools.partial(
        pl.kernel,
        out_type=jax.ShapeDtypeStruct((_N, _D), jnp.float32),
        mesh=_sc_mesh(),
        scratch_types=[
            pltpu.VMEM((_CNCH, 2 * _CCT), jnp.int32),
            pltpu.VMEM((_TOK_W, 32), jnp.float32),
            pltpu.VMEM((2, 2 * _CCT, _D), jnp.float32),
            pltpu.VMEM((2, _CCT, _D), jnp.float32),
            pltpu.SemaphoreType.DMA,
            pltpu.SemaphoreType.DMA,
        ],
    )
    def k(y_hbm, idx_hbm, w_hbm, out_hbm, idx_v, w_v, rows_v, out_v,
          gsem, ssem):
        wid = lax.axis_index("s") * _NC + lax.axis_index("c")
        base = wid * _TOK_W
        pltpu.sync_copy(idx_hbm.at[wid], idx_v)
        pltpu.sync_copy(w_hbm.at[wid], w_v)
        g = [None] * _CNCH
        st = [None] * _CNCH
        g[0] = pltpu.async_copy(y_hbm.at[idx_v.at[0]], rows_v.at[0], gsem)
        for ch in range(_CNCH):
            b = ch & 1
            if ch + 1 < _CNCH:
                if ch >= 1:
                    st[ch - 1].wait()
                g[ch + 1] = pltpu.async_copy(
                    y_hbm.at[idx_v.at[ch + 1]], rows_v.at[(ch + 1) & 1], gsem)
            g[ch].wait()

            def tok_body(i, _, b=b, ch=ch):
                w0 = w_v[ch * _CCT + i, pl.ds(0, 16)]
                w1 = w_v[ch * _CCT + i, pl.ds(16, 16)]
                for l in range(_D // 16):
                    sl = pl.ds(l * 16, 16)
                    out_v[b, i, sl] = (w0 * rows_v[b, 2 * i, sl]
                                       + w1 * rows_v[b, 2 * i + 1, sl])
                return 0

            lax.fori_loop(0, _CCT, tok_body, 0)
            st[ch] = pltpu.async_copy(
                out_v.at[b], out_hbm.at[pl.ds(base + ch * _CCT, _CCT)], ssem)
        st[_CNCH - 2].wait()
        st[_CNCH - 1].wait()

    return k(y_sorted, cidx, cw)


def kernel(x, gate_w, w1, b1, w2, b2):
    B, T, D = x.shape
    x2d = x.reshape(_N, _D)
    wts, posk, psum, bexp = _router(x2d, gate_w)

    avg = psum[0] / _N
    aux = jnp.mean((avg - 1.0 / _E) ** 2) * _E

    pos2 = posk[:, :_K]
    pidx = (pos2.reshape(_NW, _DNCH, _CT, _K)
            .transpose(0, 1, 3, 2).reshape(_NW, 2 * _DNCH, _CT))
    cidx = pos2.reshape(_NW, _CNCH, 2 * _CCT)
    cw = wts.reshape(_NW, _TOK_W, 32)

    x_sorted = _sc_dispatch_call(x2d, pidx)
    y_sorted = _ffn(x_sorted, w1, b1, w2, b2, bexp.reshape(_NBLK))
    out2d = _sc_combine_call(y_sorted, cidx, cw)
    return out2d.reshape(B, T, D), aux


# speedup vs baseline: 6.0708x; 1.0473x over previous
import functools

import jax
import jax.numpy as jnp
from jax import lax
from jax.experimental import pallas as pl
from jax.experimental.pallas import tpu as pltpu
from jax.experimental.pallas import tpu_sc as plsc

_E = 8
_K = 2
_D = 1024
_F = 4096
_N = 4096
_BM = 256
_NBLK = 40
_PAD = _BM * _NBLK
_TB = 512
_NTB = _N // _TB

_NC = 2
_NS = 16
_NW = _NC * _NS
_TOK_W = _N // _NW
_CT = 32
_DNCH = _TOK_W // _CT
_CCT = 16
_CNCH = _TOK_W // _CCT


def _top2(probs, col):
    m1 = jnp.max(probs, axis=1, keepdims=True)
    i1 = jnp.min(jnp.where(probs == m1, col, _E), axis=1, keepdims=True)
    pm = jnp.where(col == i1, -1.0, probs)
    m2 = jnp.max(pm, axis=1, keepdims=True)
    i2 = jnp.min(jnp.where(pm == m2, col, _E), axis=1, keepdims=True)
    return m1, i1, m2, i2


def _router_body(x_ref, gw_ref, w_ref, pos_ref, ps_ref, be_ref, cnt_ref,
                 pr_ref):
    p = pl.program_id(0)
    blk = pl.program_id(1)
    col = lax.broadcasted_iota(jnp.int32, (_TB, _E), 1)

    @pl.when(p == 0)
    def _pass1_probs():
        x = x_ref[...]
        gw = gw_ref[...]
        logits = lax.dot_general(x, gw, (((1,), (1,)), ((), ())),
                                 preferred_element_type=jnp.float32)
        m = jnp.max(logits, axis=1, keepdims=True)
        ex = jnp.exp(logits - m)
        pr_ref[pl.ds(blk * _TB, _TB), :] = ex / jnp.sum(ex, axis=1,
                                                        keepdims=True)

    probs = pr_ref[pl.ds(blk * _TB, _TB), :]
    m1, i1, m2, i2 = _top2(probs, col)
    oh1 = jnp.where(col == i1, 1.0, 0.0)
    oh2 = jnp.where(col == i2, 1.0, 0.0)

    @pl.when(p == 0)
    def _pass1():
        @pl.when(blk == 0)
        def _():
            ps_ref[...] = jnp.zeros_like(ps_ref)

        ps_ref[...] += jnp.sum(probs, axis=0, keepdims=True)
        cnt_ref[blk, :] = jnp.sum(oh1 + oh2, axis=0)

    @pl.when(p == 1)
    def _pass2():
        s = m1 + m2 + 1e-9
        col32 = lax.broadcasted_iota(jnp.int32, (_TB, 2 * 16), 1)
        w_ref[...] = jnp.where(col32 < 16, m1 / s, m2 / s)
        cnt = cnt_ref[...]
        totals = jnp.sum(cnt, axis=0, keepdims=True)
        blocks_e = jnp.floor((totals + (_BM - 1)) * (1.0 / _BM))
        ecol = lax.broadcasted_iota(jnp.int32, (_E, _E), 1)
        erow = lax.broadcasted_iota(jnp.int32, (_E, _E), 0)
        tri_inc = jnp.where(erow <= ecol, 1.0, 0.0)
        cumb = lax.dot_general(blocks_e, tri_inc,
                               (((1,), (0,)), ((), ())),
                               preferred_element_type=jnp.float32)
        bstart = cumb - blocks_e
        brow = lax.broadcasted_iota(jnp.int32, (_NTB, _E), 0)
        bprefix = jnp.sum(jnp.where(brow < blk, cnt, 0.0), axis=0,
                          keepdims=True)
        trow = lax.broadcasted_iota(jnp.int32, (_TB, _TB), 0)
        tcol = lax.broadcasted_iota(jnp.int32, (_TB, _TB), 1)
        ltri = jnp.where(tcol < trow, 1.0, 0.0)
        pfx = lax.dot_general(ltri, oh1 + oh2, (((1,), (0,)), ((), ())),
                              preferred_element_type=jnp.float32)
        base = bstart * _BM + bprefix + pfx
        pos0 = jnp.sum(oh1 * base, axis=1, keepdims=True)
        pos1 = jnp.sum(oh2 * base, axis=1, keepdims=True)
        pos_ref[...] = jnp.where(col == 0, pos0,
                                 jnp.where(col == 1, pos1, 0.0)
                                 ).astype(jnp.int32)

        @pl.when(blk == 0)
        def _():
            bro = lax.broadcasted_iota(jnp.int32, (_NBLK + 1, _E), 0
                                       ).astype(jnp.float32)
            be = jnp.minimum(jnp.sum(jnp.where(cumb <= bro, 1.0, 0.0),
                                     axis=1, keepdims=True), _E - 1)
            lane = lax.broadcasted_iota(jnp.int32, (1, _E), 1)
            tot = jnp.sum(jnp.where(lane == _E - 1, cumb, 0.0), axis=1,
                          keepdims=True)
            rowi = lax.broadcasted_iota(jnp.int32, (_NBLK + 1, 1), 0)
            be_ref[...] = jnp.where(rowi < _NBLK, be, tot).astype(jnp.int32)


def _router(x2d, gate_w):
    return pl.pallas_call(
        _router_body,
        grid=(2, _NTB),
        in_specs=[
            pl.BlockSpec((_TB, _D), lambda p, b: (b, 0)),
            pl.BlockSpec((_E, _D), lambda p, b: (0, 0)),
        ],
        out_specs=[
            pl.BlockSpec((_TB, 32), lambda p, b: (b, 0)),
            pl.BlockSpec((_TB, _E), lambda p, b: (b, 0)),
            pl.BlockSpec((1, _E), lambda p, b: (0, 0)),
            pl.BlockSpec((_NBLK + 1, 1), lambda p, b: (0, 0)),
        ],
        out_shape=[
            jax.ShapeDtypeStruct((_N, 32), jnp.float32),
            jax.ShapeDtypeStruct((_N, _E), jnp.int32),
            jax.ShapeDtypeStruct((1, _E), jnp.float32),
            jax.ShapeDtypeStruct((_NBLK + 1, 1), jnp.int32),
        ],
        scratch_shapes=[pltpu.VMEM((_NTB, _E), jnp.float32),
                        pltpu.VMEM((_N, _E), jnp.float32)],
    )(x2d, gate_w)


def _ffn_body(be_ref, x_ref, w1_ref, b1_ref, w2_ref, b2_ref, out_ref):
    @pl.when(pl.program_id(0) < be_ref[_NBLK])
    def _():
        x = x_ref[...].astype(jnp.bfloat16)
        h = lax.dot_general(x, w1_ref[0], (((1,), (1,)), ((), ())),
                            preferred_element_type=jnp.float32)
        h = h + b1_ref[0]
        h = 0.5 * h * (1.0 + lax.erf(h * (2.0 ** -0.5)))
        y = lax.dot_general(h.astype(jnp.bfloat16), w2_ref[0],
                            (((1,), (1,)), ((), ())),
                            preferred_element_type=jnp.float32)
        out_ref[...] = y + b2_ref[0]


def _ffn(x_sorted, w1, b1, w2, b2, block_expert):
    grid_spec = pltpu.PrefetchScalarGridSpec(
        num_scalar_prefetch=1,
        grid=(_NBLK,),
        in_specs=[
            pl.BlockSpec((_BM, _D), lambda i, be: (i, 0)),
            pl.BlockSpec((1, _F, _D), lambda i, be: (be[i], 0, 0)),
            pl.BlockSpec((1, 1, _F), lambda i, be: (be[i], 0, 0)),
            pl.BlockSpec((1, _D, _F), lambda i, be: (be[i], 0, 0)),
            pl.BlockSpec((1, 1, _D), lambda i, be: (be[i], 0, 0)),
        ],
        out_specs=pl.BlockSpec((_BM, _D), lambda i, be: (i, 0)),
    )
    return pl.pallas_call(
        _ffn_body,
        grid_spec=grid_spec,
        out_shape=jax.ShapeDtypeStruct((_PAD, _D), jnp.float32),
    )(block_expert, x_sorted, w1.astype(jnp.bfloat16),
      b1.reshape(_E, 1, _F), w2.astype(jnp.bfloat16),
      b2.reshape(_E, 1, _D))


def _sc_mesh():
    return plsc.VectorSubcoreMesh(core_axis_name="c", subcore_axis_name="s",
                                  num_cores=_NC, num_subcores=_NS)


def _sc_dispatch_call(x2d, pidx):
    @functools.partial(
        pl.kernel,
        out_type=jax.ShapeDtypeStruct((_PAD, _D), jnp.float32),
        mesh=_sc_mesh(),
        scratch_types=[
            pltpu.VMEM((2 * _DNCH, _CT), jnp.int32),
            pltpu.VMEM((2, _CT, _D), jnp.float32),
            pltpu.SemaphoreType.DMA,
            pltpu.SemaphoreType.DMA,
        ],
    )
    def k(x_hbm, idx_hbm, out_hbm, idx_v, rows_v, lsem, ssem):
        wid = lax.axis_index("s") * _NC + lax.axis_index("c")
        tbase = wid * _TOK_W
        pltpu.sync_copy(idx_hbm.at[wid], idx_v)
        ld = [None] * _DNCH
        st = [[None, None] for _ in range(_DNCH)]
        ld[0] = pltpu.async_copy(
            x_hbm.at[pl.ds(tbase, _CT)], rows_v.at[0], lsem)
        for ch in range(_DNCH):
            b = ch & 1
            if ch + 1 < _DNCH:
                if ch >= 1:
                    st[ch - 1][0].wait()
                    st[ch - 1][1].wait()
                ld[ch + 1] = pltpu.async_copy(
                    x_hbm.at[pl.ds(tbase + (ch + 1) * _CT, _CT)],
                    rows_v.at[(ch + 1) & 1], lsem)
            ld[ch].wait()
            st[ch][0] = pltpu.async_copy(
                rows_v.at[b], out_hbm.at[idx_v.at[2 * ch]], ssem)
            st[ch][1] = pltpu.async_copy(
                rows_v.at[b], out_hbm.at[idx_v.at[2 * ch + 1]], ssem)
        for ch in (_DNCH - 2, _DNCH - 1):
            st[ch][0].wait()
            st[ch][1].wait()

    return k(x2d, pidx)


def _sc_combine_call(y_sorted, cidx, cw):
    @functools.partial(
        pl.kernel,
        out_type=jax.ShapeDtypeStruct((_N, _D), jnp.float32),
        mesh=_sc_mesh(),
        scratch_types=[
            pltpu.VMEM((_CNCH, 2 * _CCT), jnp.int32),
            pltpu.VMEM((_TOK_W, 32), jnp.float32),
            pltpu.VMEM((2, 2 * _CCT, _D), jnp.float32),
            pltpu.VMEM((2, _CCT, _D), jnp.float32),
            pltpu.SemaphoreType.DMA,
            pltpu.SemaphoreType.DMA,
        ],
    )
    def k(y_hbm, idx_hbm, w_hbm, out_hbm, idx_v, w_v, rows_v, out_v,
          gsem, ssem):
        wid = lax.axis_index("s") * _NC + lax.axis_index("c")
        base = wid * _TOK_W
        pltpu.sync_copy(idx_hbm.at[wid], idx_v)
        pltpu.sync_copy(w_hbm.at[wid], w_v)
        g = [None] * _CNCH
        st = [None] * _CNCH
        g[0] = pltpu.async_copy(y_hbm.at[idx_v.at[0]], rows_v.at[0], gsem)
        for ch in range(_CNCH):
            b = ch & 1
            if ch + 1 < _CNCH:
                if ch >= 1:
                    st[ch - 1].wait()
                g[ch + 1] = pltpu.async_copy(
                    y_hbm.at[idx_v.at[ch + 1]], rows_v.at[(ch + 1) & 1], gsem)
            g[ch].wait()

            def tok_body(i, _, b=b, ch=ch):
                w0 = w_v[ch * _CCT + i, pl.ds(0, 16)]
                w1 = w_v[ch * _CCT + i, pl.ds(16, 16)]
                for l in range(_D // 16):
                    sl = pl.ds(l * 16, 16)
                    out_v[b, i, sl] = (w0 * rows_v[b, 2 * i, sl]
                                       + w1 * rows_v[b, 2 * i + 1, sl])
                return 0

            lax.fori_loop(0, _CCT, tok_body, 0)
            st[ch] = pltpu.async_copy(
                out_v.at[b], out_hbm.at[pl.ds(base + ch * _CCT, _CCT)], ssem)
        st[_CNCH - 2].wait()
        st[_CNCH - 1].wait()

    return k(y_sorted, cidx, cw)


def kernel(x, gate_w, w1, b1, w2, b2):
    B, T, D = x.shape
    x2d = x.reshape(_N, _D)
    wts, posk, psum, bexp = _router(x2d, gate_w)

    avg = psum[0] / _N
    aux = jnp.mean((avg - 1.0 / _E) ** 2) * _E

    pos2 = posk[:, :_K]
    pidx = (pos2.reshape(_NW, _DNCH, _CT, _K)
            .transpose(0, 1, 3, 2).reshape(_NW, 2 * _DNCH, _CT))
    cidx = pos2.reshape(_NW, _CNCH, 2 * _CCT)
    cw = wts.reshape(_NW, _TOK_W, 32)

    x_sorted = _sc_dispatch_call(x2d, pidx)
    y_sorted = _ffn(x_sorted, w1, b1, w2, b2, bexp.reshape(_NBLK + 1))
    out2d = _sc_combine_call(y_sorted, cidx, cw)
    return out2d.reshape(B, T, D), aux


# speedup vs baseline: 6.0950x; 1.0040x over previous
import functools

import jax
import jax.numpy as jnp
from jax import lax
from jax.experimental import pallas as pl
from jax.experimental.pallas import tpu as pltpu
from jax.experimental.pallas import tpu_sc as plsc

_E = 8
_K = 2
_D = 1024
_F = 4096
_N = 4096
_BM = 256
_NBLK = 40
_PAD = _BM * _NBLK
_TB = 512
_NTB = _N // _TB

_NC = 2
_NS = 16
_NW = _NC * _NS
_TOK_W = _N // _NW
_CT = 32
_DNCH = _TOK_W // _CT
_CCT = 8
_CNCH = _TOK_W // _CCT


def _top2(probs, col):
    m1 = jnp.max(probs, axis=1, keepdims=True)
    i1 = jnp.min(jnp.where(probs == m1, col, _E), axis=1, keepdims=True)
    pm = jnp.where(col == i1, -1.0, probs)
    m2 = jnp.max(pm, axis=1, keepdims=True)
    i2 = jnp.min(jnp.where(pm == m2, col, _E), axis=1, keepdims=True)
    return m1, i1, m2, i2


def _router_body(x_ref, gw_ref, w_ref, pos_ref, ps_ref, be_ref, cnt_ref,
                 pr_ref):
    p = pl.program_id(0)
    blk = pl.program_id(1)
    col = lax.broadcasted_iota(jnp.int32, (_TB, _E), 1)

    @pl.when(p == 0)
    def _pass1_probs():
        x = x_ref[...]
        gw = gw_ref[...]
        logits = lax.dot_general(x, gw, (((1,), (1,)), ((), ())),
                                 preferred_element_type=jnp.float32)
        m = jnp.max(logits, axis=1, keepdims=True)
        ex = jnp.exp(logits - m)
        pr_ref[pl.ds(blk * _TB, _TB), :] = ex / jnp.sum(ex, axis=1,
                                                        keepdims=True)

    probs = pr_ref[pl.ds(blk * _TB, _TB), :]
    m1, i1, m2, i2 = _top2(probs, col)
    oh1 = jnp.where(col == i1, 1.0, 0.0)
    oh2 = jnp.where(col == i2, 1.0, 0.0)

    @pl.when(p == 0)
    def _pass1():
        @pl.when(blk == 0)
        def _():
            ps_ref[...] = jnp.zeros_like(ps_ref)

        ps_ref[...] += jnp.sum(probs, axis=0, keepdims=True)
        cnt_ref[blk, :] = jnp.sum(oh1 + oh2, axis=0)

    @pl.when(p == 1)
    def _pass2():
        s = m1 + m2 + 1e-9
        col32 = lax.broadcasted_iota(jnp.int32, (_TB, 2 * 16), 1)
        w_ref[...] = jnp.where(col32 < 16, m1 / s, m2 / s)
        cnt = cnt_ref[...]
        totals = jnp.sum(cnt, axis=0, keepdims=True)
        blocks_e = jnp.floor((totals + (_BM - 1)) * (1.0 / _BM))
        ecol = lax.broadcasted_iota(jnp.int32, (_E, _E), 1)
        erow = lax.broadcasted_iota(jnp.int32, (_E, _E), 0)
        tri_inc = jnp.where(erow <= ecol, 1.0, 0.0)
        cumb = lax.dot_general(blocks_e, tri_inc,
                               (((1,), (0,)), ((), ())),
                               preferred_element_type=jnp.float32)
        bstart = cumb - blocks_e
        brow = lax.broadcasted_iota(jnp.int32, (_NTB, _E), 0)
        bprefix = jnp.sum(jnp.where(brow < blk, cnt, 0.0), axis=0,
                          keepdims=True)
        trow = lax.broadcasted_iota(jnp.int32, (_TB, _TB), 0)
        tcol = lax.broadcasted_iota(jnp.int32, (_TB, _TB), 1)
        ltri = jnp.where(tcol < trow, 1.0, 0.0)
        pfx = lax.dot_general(ltri, oh1 + oh2, (((1,), (0,)), ((), ())),
                              preferred_element_type=jnp.float32)
        base = bstart * _BM + bprefix + pfx
        pos0 = jnp.sum(oh1 * base, axis=1, keepdims=True)
        pos1 = jnp.sum(oh2 * base, axis=1, keepdims=True)
        pos_ref[...] = jnp.where(col == 0, pos0,
                                 jnp.where(col == 1, pos1, 0.0)
                                 ).astype(jnp.int32)

        @pl.when(blk == 0)
        def _():
            bro = lax.broadcasted_iota(jnp.int32, (_NBLK + 1, _E), 0
                                       ).astype(jnp.float32)
            be = jnp.minimum(jnp.sum(jnp.where(cumb <= bro, 1.0, 0.0),
                                     axis=1, keepdims=True), _E - 1)
            lane = lax.broadcasted_iota(jnp.int32, (1, _E), 1)
            tot = jnp.sum(jnp.where(lane == _E - 1, cumb, 0.0), axis=1,
                          keepdims=True)
            rowi = lax.broadcasted_iota(jnp.int32, (_NBLK + 1, 1), 0)
            be_ref[...] = jnp.where(rowi < _NBLK, be, tot).astype(jnp.int32)


def _router(x2d, gate_w):
    return pl.pallas_call(
        _router_body,
        grid=(2, _NTB),
        in_specs=[
            pl.BlockSpec((_TB, _D), lambda p, b: (b, 0)),
            pl.BlockSpec((_E, _D), lambda p, b: (0, 0)),
        ],
        out_specs=[
            pl.BlockSpec((_TB, 32), lambda p, b: (b, 0)),
            pl.BlockSpec((_TB, _E), lambda p, b: (b, 0)),
            pl.BlockSpec((1, _E), lambda p, b: (0, 0)),
            pl.BlockSpec((_NBLK + 1, 1), lambda p, b: (0, 0)),
        ],
        out_shape=[
            jax.ShapeDtypeStruct((_N, 32), jnp.float32),
            jax.ShapeDtypeStruct((_N, _E), jnp.int32),
            jax.ShapeDtypeStruct((1, _E), jnp.float32),
            jax.ShapeDtypeStruct((_NBLK + 1, 1), jnp.int32),
        ],
        scratch_shapes=[pltpu.VMEM((_NTB, _E), jnp.float32),
                        pltpu.VMEM((_N, _E), jnp.float32)],
    )(x2d, gate_w)


def _ffn_body(be_ref, x_ref, w1_ref, b1_ref, w2_ref, b2_ref, out_ref,
              w1b, w2b):
    i = pl.program_id(1)
    j = pl.program_id(0)
    prev = be_ref[jnp.maximum(i - 1, 0)]
    changed = jnp.logical_or(i == 0, be_ref[i] != prev)

    @pl.when(jnp.logical_and(i < be_ref[_NBLK], changed))
    def _():
        w1b[...] = w1_ref[0].astype(jnp.bfloat16)
        w2b[...] = w2_ref[0].astype(jnp.bfloat16)

    @pl.when(i < be_ref[_NBLK])
    def _():
        x = x_ref[...].astype(jnp.bfloat16)
        h = lax.dot_general(x, w1b[...], (((1,), (1,)), ((), ())),
                            preferred_element_type=jnp.float32)
        h = h + b1_ref[0]
        h = 0.5 * h * (1.0 + lax.erf(h * (2.0 ** -0.5)))
        y = lax.dot_general(h.astype(jnp.bfloat16), w2b[...],
                            (((1,), (1,)), ((), ())),
                            preferred_element_type=jnp.float32)
        out_ref[0] = jnp.where(j == 1, y + b2_ref[0], y)


_FH = _F // 2


def _ffn(x_sorted, w1, b1, w2, b2, block_expert):
    grid_spec = pltpu.PrefetchScalarGridSpec(
        num_scalar_prefetch=1,
        grid=(2, _NBLK),
        in_specs=[
            pl.BlockSpec((_BM, _D), lambda j, i, be: (i, 0)),
            pl.BlockSpec((1, _FH, _D), lambda j, i, be: (be[i], j, 0)),
            pl.BlockSpec((1, 1, _FH), lambda j, i, be: (be[i], 0, j)),
            pl.BlockSpec((1, _D, _FH), lambda j, i, be: (be[i], 0, j)),
            pl.BlockSpec((1, 1, _D), lambda j, i, be: (be[i], 0, 0)),
        ],
        out_specs=pl.BlockSpec((1, _BM, _D), lambda j, i, be: (j, i, 0)),
        scratch_shapes=[pltpu.VMEM((_FH, _D), jnp.bfloat16),
                        pltpu.VMEM((_D, _FH), jnp.bfloat16)],
    )
    return pl.pallas_call(
        _ffn_body,
        grid_spec=grid_spec,
        out_shape=jax.ShapeDtypeStruct((2, _PAD, _D), jnp.float32),
    )(block_expert, x_sorted, w1, b1.reshape(_E, 1, _F), w2,
      b2.reshape(_E, 1, _D))


def _sc_mesh():
    return plsc.VectorSubcoreMesh(core_axis_name="c", subcore_axis_name="s",
                                  num_cores=_NC, num_subcores=_NS)


def _sc_dispatch_call(x2d, pidx):
    @functools.partial(
        pl.kernel,
        out_type=jax.ShapeDtypeStruct((_PAD, _D), jnp.float32),
        mesh=_sc_mesh(),
        scratch_types=[
            pltpu.VMEM((2 * _DNCH, _CT), jnp.int32),
            pltpu.VMEM((2, _CT, _D), jnp.float32),
            pltpu.SemaphoreType.DMA,
            pltpu.SemaphoreType.DMA,
        ],
    )
    def k(x_hbm, idx_hbm, out_hbm, idx_v, rows_v, lsem, ssem):
        wid = lax.axis_index("s") * _NC + lax.axis_index("c")
        tbase = wid * _TOK_W
        pltpu.sync_copy(idx_hbm.at[wid], idx_v)
        ld = [None] * _DNCH
        st = [[None, None] for _ in range(_DNCH)]
        ld[0] = pltpu.async_copy(
            x_hbm.at[pl.ds(tbase, _CT)], rows_v.at[0], lsem)
        for ch in range(_DNCH):
            b = ch & 1
            if ch + 1 < _DNCH:
                if ch >= 1:
                    st[ch - 1][0].wait()
                    st[ch - 1][1].wait()
                ld[ch + 1] = pltpu.async_copy(
                    x_hbm.at[pl.ds(tbase + (ch + 1) * _CT, _CT)],
                    rows_v.at[(ch + 1) & 1], lsem)
            ld[ch].wait()
            st[ch][0] = pltpu.async_copy(
                rows_v.at[b], out_hbm.at[idx_v.at[2 * ch]], ssem)
            st[ch][1] = pltpu.async_copy(
                rows_v.at[b], out_hbm.at[idx_v.at[2 * ch + 1]], ssem)
        for ch in (_DNCH - 2, _DNCH - 1):
            st[ch][0].wait()
            st[ch][1].wait()

    return k(x2d, pidx)


def _sc_combine_call(y_sorted, cidx, cw):
    @functools.partial(
        pl.kernel,
        out_type=jax.ShapeDtypeStruct((_N, _D), jnp.float32),
        mesh=_sc_mesh(),
        scratch_types=[
            pltpu.VMEM((_CNCH, 4 * _CCT), jnp.int32),
            pltpu.VMEM((_TOK_W, 32), jnp.float32),
            pltpu.VMEM((2, 4 * _CCT, _D), jnp.float32),
            pltpu.VMEM((2, _CCT, _D), jnp.float32),
            pltpu.SemaphoreType.DMA,
            pltpu.SemaphoreType.DMA,
        ],
    )
    def k(y_hbm, idx_hbm, w_hbm, out_hbm, idx_v, w_v, rows_v, out_v,
          gsem, ssem):
        wid = lax.axis_index("s") * _NC + lax.axis_index("c")
        base = wid * _TOK_W
        pltpu.sync_copy(idx_hbm.at[wid], idx_v)
        pltpu.sync_copy(w_hbm.at[wid], w_v)

        def chunk_compute(ch, b):
            def tok_body(i, _):
                w0 = w_v[ch * _CCT + i, pl.ds(0, 16)]
                w1 = w_v[ch * _CCT + i, pl.ds(16, 16)]
                for l in range(_D // 16):
                    sl = pl.ds(l * 16, 16)
                    out_v[b, i, sl] = (
                        w0 * (rows_v[b, 4 * i, sl] + rows_v[b, 4 * i + 1, sl])
                        + w1 * (rows_v[b, 4 * i + 2, sl]
                                + rows_v[b, 4 * i + 3, sl]))
                return 0

            lax.fori_loop(0, _CCT, tok_body, 0)

        def pair_body(it, _):
            c0 = 2 * it
            g0 = pltpu.async_copy(y_hbm.at[idx_v.at[c0]], rows_v.at[0], gsem)
            g1 = pltpu.async_copy(y_hbm.at[idx_v.at[c0 + 1]], rows_v.at[1],
                                  gsem)
            g0.wait()
            chunk_compute(c0, 0)
            st0 = pltpu.async_copy(
                out_v.at[0], out_hbm.at[pl.ds(base + c0 * _CCT, _CCT)], ssem)
            g1.wait()
            chunk_compute(c0 + 1, 1)
            st1 = pltpu.async_copy(
                out_v.at[1],
                out_hbm.at[pl.ds(base + (c0 + 1) * _CCT, _CCT)], ssem)
            st0.wait()
            st1.wait()
            return 0

        lax.fori_loop(0, _CNCH // 2, pair_body, 0)

    return k(y_sorted, cidx, cw)


def kernel(x, gate_w, w1, b1, w2, b2):
    B, T, D = x.shape
    x2d = x.reshape(_N, _D)
    wts, posk, psum, bexp = _router(x2d, gate_w)

    avg = psum[0] / _N
    aux = jnp.mean((avg - 1.0 / _E) ** 2) * _E

    pos2 = posk[:, :_K]
    pidx = (pos2.reshape(_NW, _DNCH, _CT, _K)
            .transpose(0, 1, 3, 2).reshape(_NW, 2 * _DNCH, _CT))
    p0, p1 = pos2[:, 0], pos2[:, 1]
    cidx = (jnp.stack([p0, p0 + _PAD, p1, p1 + _PAD], axis=1)
            .reshape(_NW, _CNCH, 4 * _CCT))
    cw = wts.reshape(_NW, _TOK_W, 32)

    x_sorted = _sc_dispatch_call(x2d, pidx)
    y2 = _ffn(x_sorted, w1, b1, w2, b2, bexp.reshape(_NBLK + 1))
    out2d = _sc_combine_call(y2.reshape(2 * _PAD, _D), cidx, cw)
    return out2d.reshape(B, T, D), aux


# speedup vs baseline: 6.1085x; 1.0022x over previous
import functools

import jax
import jax.numpy as jnp
from jax import lax
from jax.experimental import pallas as pl
from jax.experimental.pallas import tpu as pltpu
from jax.experimental.pallas import tpu_sc as plsc

_E = 8
_K = 2
_D = 1024
_F = 4096
_N = 4096
_BM = 256
_NBLK = 40
_PAD = _BM * _NBLK
_TB = 512
_NTB = _N // _TB

_NC = 2
_NS = 16
_NW = _NC * _NS
_TOK_W = _N // _NW
_CT = 32
_DNCH = _TOK_W // _CT
_CCT = 8
_CNCH = _TOK_W // _CCT


def _top2(probs, col):
    m1 = jnp.max(probs, axis=1, keepdims=True)
    i1 = jnp.min(jnp.where(probs == m1, col, _E), axis=1, keepdims=True)
    pm = jnp.where(col == i1, -1.0, probs)
    m2 = jnp.max(pm, axis=1, keepdims=True)
    i2 = jnp.min(jnp.where(pm == m2, col, _E), axis=1, keepdims=True)
    return m1, i1, m2, i2


def _router_body(x_ref, gw_ref, w_ref, pos_ref, ps_ref, be_ref, cnt_ref,
                 pr_ref):
    p = pl.program_id(0)
    blk = pl.program_id(1)
    col = lax.broadcasted_iota(jnp.int32, (_TB, _E), 1)

    @pl.when(p == 0)
    def _pass1_probs():
        x = x_ref[...]
        gw = gw_ref[...]
        logits = lax.dot_general(x, gw, (((1,), (1,)), ((), ())),
                                 preferred_element_type=jnp.float32)
        m = jnp.max(logits, axis=1, keepdims=True)
        ex = jnp.exp(logits - m)
        pr_ref[pl.ds(blk * _TB, _TB), :] = ex / jnp.sum(ex, axis=1,
                                                        keepdims=True)

    probs = pr_ref[pl.ds(blk * _TB, _TB), :]
    m1, i1, m2, i2 = _top2(probs, col)
    oh1 = jnp.where(col == i1, 1.0, 0.0)
    oh2 = jnp.where(col == i2, 1.0, 0.0)

    @pl.when(p == 0)
    def _pass1():
        @pl.when(blk == 0)
        def _():
            ps_ref[...] = jnp.zeros_like(ps_ref)

        ps_ref[...] += jnp.sum(probs, axis=0, keepdims=True)
        cnt_ref[blk, :] = jnp.sum(oh1 + oh2, axis=0)

    @pl.when(p == 1)
    def _pass2():
        s = m1 + m2 + 1e-9
        col32 = lax.broadcasted_iota(jnp.int32, (_TB, 2 * 16), 1)
        w_ref[...] = jnp.where(col32 < 16, m1 / s, m2 / s)
        cnt = cnt_ref[...]
        totals = jnp.sum(cnt, axis=0, keepdims=True)
        blocks_e = jnp.floor((totals + (_BM - 1)) * (1.0 / _BM))
        ecol = lax.broadcasted_iota(jnp.int32, (_E, _E), 1)
        erow = lax.broadcasted_iota(jnp.int32, (_E, _E), 0)
        tri_inc = jnp.where(erow <= ecol, 1.0, 0.0)
        cumb = lax.dot_general(blocks_e, tri_inc,
                               (((1,), (0,)), ((), ())),
                               preferred_element_type=jnp.float32)
        bstart = cumb - blocks_e
        brow = lax.broadcasted_iota(jnp.int32, (_NTB, _E), 0)
        bprefix = jnp.sum(jnp.where(brow < blk, cnt, 0.0), axis=0,
                          keepdims=True)
        trow = lax.broadcasted_iota(jnp.int32, (_TB, _TB), 0)
        tcol = lax.broadcasted_iota(jnp.int32, (_TB, _TB), 1)
        ltri = jnp.where(tcol < trow, 1.0, 0.0)
        pfx = lax.dot_general(ltri, oh1 + oh2, (((1,), (0,)), ((), ())),
                              preferred_element_type=jnp.float32)
        base = bstart * _BM + bprefix + pfx
        pos0 = jnp.sum(oh1 * base, axis=1, keepdims=True)
        pos1 = jnp.sum(oh2 * base, axis=1, keepdims=True)
        pos_ref[...] = jnp.where(col == 0, pos0,
                                 jnp.where(col == 1, pos1, 0.0)
                                 ).astype(jnp.int32)

        @pl.when(blk == 0)
        def _():
            bro = lax.broadcasted_iota(jnp.int32, (_NBLK + 1, _E), 0
                                       ).astype(jnp.float32)
            be = jnp.minimum(jnp.sum(jnp.where(cumb <= bro, 1.0, 0.0),
                                     axis=1, keepdims=True), _E - 1)
            lane = lax.broadcasted_iota(jnp.int32, (1, _E), 1)
            tot = jnp.sum(jnp.where(lane == _E - 1, cumb, 0.0), axis=1,
                          keepdims=True)
            rowi = lax.broadcasted_iota(jnp.int32, (_NBLK + 1, 1), 0)
            be_ref[...] = jnp.where(rowi < _NBLK, be, tot).astype(jnp.int32)


def _router(x2d, gate_w):
    return pl.pallas_call(
        _router_body,
        grid=(2, _NTB),
        in_specs=[
            pl.BlockSpec((_TB, _D), lambda p, b: (b * (1 - p), 0)),
            pl.BlockSpec((_E, _D), lambda p, b: (0, 0)),
        ],
        out_specs=[
            pl.BlockSpec((_TB, 32), lambda p, b: (b, 0)),
            pl.BlockSpec((_TB, _E), lambda p, b: (b, 0)),
            pl.BlockSpec((1, _E), lambda p, b: (0, 0)),
            pl.BlockSpec((_NBLK + 1, 1), lambda p, b: (0, 0)),
        ],
        out_shape=[
            jax.ShapeDtypeStruct((_N, 32), jnp.float32),
            jax.ShapeDtypeStruct((_N, _E), jnp.int32),
            jax.ShapeDtypeStruct((1, _E), jnp.float32),
            jax.ShapeDtypeStruct((_NBLK + 1, 1), jnp.int32),
        ],
        scratch_shapes=[pltpu.VMEM((_NTB, _E), jnp.float32),
                        pltpu.VMEM((_N, _E), jnp.float32)],
    )(x2d, gate_w)


def _ffn_body(be_ref, x_ref, w1_ref, b1_ref, w2_ref, b2_ref, out_ref,
              w1b, w2b):
    i = pl.program_id(1)
    j = pl.program_id(0)
    prev = be_ref[jnp.maximum(i - 1, 0)]
    changed = jnp.logical_or(i == 0, be_ref[i] != prev)

    @pl.when(jnp.logical_and(i < be_ref[_NBLK], changed))
    def _():
        w1b[...] = w1_ref[0].astype(jnp.bfloat16)
        w2b[...] = w2_ref[0].astype(jnp.bfloat16)

    @pl.when(i < be_ref[_NBLK])
    def _():
        x = x_ref[...].astype(jnp.bfloat16)
        h = lax.dot_general(x, w1b[...], (((1,), (1,)), ((), ())),
                            preferred_element_type=jnp.float32)
        h = h + b1_ref[0]
        h = 0.5 * h * (1.0 + lax.erf(h * (2.0 ** -0.5)))
        y = lax.dot_general(h.astype(jnp.bfloat16), w2b[...],
                            (((1,), (1,)), ((), ())),
                            preferred_element_type=jnp.float32)
        out_ref[0] = jnp.where(j == 1, y + b2_ref[0], y)


_FH = _F // 2


def _ffn(x_sorted, w1, b1, w2, b2, block_expert):
    grid_spec = pltpu.PrefetchScalarGridSpec(
        num_scalar_prefetch=1,
        grid=(2, _NBLK),
        in_specs=[
            pl.BlockSpec((_BM, _D), lambda j, i, be: (i, 0)),
            pl.BlockSpec((1, _FH, _D), lambda j, i, be: (be[i], j, 0)),
            pl.BlockSpec((1, 1, _FH), lambda j, i, be: (be[i], 0, j)),
            pl.BlockSpec((1, _D, _FH), lambda j, i, be: (be[i], 0, j)),
            pl.BlockSpec((1, 1, _D), lambda j, i, be: (be[i], 0, 0)),
        ],
        out_specs=pl.BlockSpec((1, _BM, _D), lambda j, i, be: (j, i, 0)),
        scratch_shapes=[pltpu.VMEM((_FH, _D), jnp.bfloat16),
                        pltpu.VMEM((_D, _FH), jnp.bfloat16)],
    )
    return pl.pallas_call(
        _ffn_body,
        grid_spec=grid_spec,
        out_shape=jax.ShapeDtypeStruct((2, _PAD, _D), jnp.float32),
    )(block_expert, x_sorted, w1, b1.reshape(_E, 1, _F), w2,
      b2.reshape(_E, 1, _D))


def _sc_mesh():
    return plsc.VectorSubcoreMesh(core_axis_name="c", subcore_axis_name="s",
                                  num_cores=_NC, num_subcores=_NS)


def _sc_dispatch_call(x2d, pidx):
    @functools.partial(
        pl.kernel,
        out_type=jax.ShapeDtypeStruct((_PAD, _D), jnp.float32),
        mesh=_sc_mesh(),
        scratch_types=[
            pltpu.VMEM((2 * _DNCH, _CT), jnp.int32),
            pltpu.VMEM((2, _CT, _D), jnp.float32),
            pltpu.SemaphoreType.DMA,
            pltpu.SemaphoreType.DMA,
        ],
    )
    def k(x_hbm, idx_hbm, out_hbm, idx_v, rows_v, lsem, ssem):
        wid = lax.axis_index("s") * _NC + lax.axis_index("c")
        tbase = wid * _TOK_W
        pltpu.sync_copy(idx_hbm.at[wid], idx_v)
        ld = [None] * _DNCH
        st = [[None, None] for _ in range(_DNCH)]
        ld[0] = pltpu.async_copy(
            x_hbm.at[pl.ds(tbase, _CT)], rows_v.at[0], lsem)
        for ch in range(_DNCH):
            b = ch & 1
            if ch + 1 < _DNCH:
                if ch >= 1:
                    st[ch - 1][0].wait()
                    st[ch - 1][1].wait()
                ld[ch + 1] = pltpu.async_copy(
                    x_hbm.at[pl.ds(tbase + (ch + 1) * _CT, _CT)],
                    rows_v.at[(ch + 1) & 1], lsem)
            ld[ch].wait()
            st[ch][0] = pltpu.async_copy(
                rows_v.at[b], out_hbm.at[idx_v.at[2 * ch]], ssem)
            st[ch][1] = pltpu.async_copy(
                rows_v.at[b], out_hbm.at[idx_v.at[2 * ch + 1]], ssem)
        for ch in (_DNCH - 2, _DNCH - 1):
            st[ch][0].wait()
            st[ch][1].wait()

    return k(x2d, pidx)


def _sc_combine_call(y_sorted, cidx, cw):
    @functools.partial(
        pl.kernel,
        out_type=jax.ShapeDtypeStruct((_N, _D), jnp.float32),
        mesh=_sc_mesh(),
        scratch_types=[
            pltpu.VMEM((_CNCH, 4 * _CCT), jnp.int32),
            pltpu.VMEM((_TOK_W, 32), jnp.float32),
            pltpu.VMEM((2, 4 * _CCT, _D), jnp.float32),
            pltpu.VMEM((2, _CCT, _D), jnp.float32),
            pltpu.SemaphoreType.DMA,
            pltpu.SemaphoreType.DMA,
        ],
    )
    def k(y_hbm, idx_hbm, w_hbm, out_hbm, idx_v, w_v, rows_v, out_v,
          gsem, ssem):
        wid = lax.axis_index("s") * _NC + lax.axis_index("c")
        base = wid * _TOK_W
        pltpu.sync_copy(idx_hbm.at[wid], idx_v)
        pltpu.sync_copy(w_hbm.at[wid], w_v)

        def chunk_compute(ch, b):
            def tok_body(i, _):
                w0 = w_v[ch * _CCT + i, pl.ds(0, 16)]
                w1 = w_v[ch * _CCT + i, pl.ds(16, 16)]
                for l in range(_D // 16):
                    sl = pl.ds(l * 16, 16)
                    out_v[b, i, sl] = (
                        w0 * (rows_v[b, 4 * i, sl] + rows_v[b, 4 * i + 1, sl])
                        + w1 * (rows_v[b, 4 * i + 2, sl]
                                + rows_v[b, 4 * i + 3, sl]))
                return 0

            lax.fori_loop(0, _CCT, tok_body, 0)

        def pair_body(it, _):
            c0 = 2 * it
            g0 = pltpu.async_copy(y_hbm.at[idx_v.at[c0]], rows_v.at[0], gsem)
            g1 = pltpu.async_copy(y_hbm.at[idx_v.at[c0 + 1]], rows_v.at[1],
                                  gsem)
            g0.wait()
            chunk_compute(c0, 0)
            st0 = pltpu.async_copy(
                out_v.at[0], out_hbm.at[pl.ds(base + c0 * _CCT, _CCT)], ssem)
            g1.wait()
            chunk_compute(c0 + 1, 1)
            st1 = pltpu.async_copy(
                out_v.at[1],
                out_hbm.at[pl.ds(base + (c0 + 1) * _CCT, _CCT)], ssem)
            st0.wait()
            st1.wait()
            return 0

        lax.fori_loop(0, _CNCH // 2, pair_body, 0)

    return k(y_sorted, cidx, cw)


def kernel(x, gate_w, w1, b1, w2, b2):
    B, T, D = x.shape
    x2d = x.reshape(_N, _D)
    wts, posk, psum, bexp = _router(x2d, gate_w)

    avg = psum[0] / _N
    aux = jnp.mean((avg - 1.0 / _E) ** 2) * _E

    pos2 = posk[:, :_K]
    pidx = (pos2.reshape(_NW, _DNCH, _CT, _K)
            .transpose(0, 1, 3, 2).reshape(_NW, 2 * _DNCH, _CT))
    p0, p1 = pos2[:, 0], pos2[:, 1]
    cidx = (jnp.stack([p0, p0 + _PAD, p1, p1 + _PAD], axis=1)
            .reshape(_NW, _CNCH, 4 * _CCT))
    cw = wts.reshape(_NW, _TOK_W, 32)

    x_sorted = _sc_dispatch_call(x2d, pidx)
    y2 = _ffn(x_sorted, w1, b1, w2, b2, bexp.reshape(_NBLK + 1))
    out2d = _sc_combine_call(y2.reshape(2 * _PAD, _D), cidx, cw)
    return out2d.reshape(B, T, D), aux


# speedup vs baseline: 6.1187x; 1.0017x over previous
import functools

import jax
import jax.numpy as jnp
from jax import lax
from jax.experimental import pallas as pl
from jax.experimental.pallas import tpu as pltpu
from jax.experimental.pallas import tpu_sc as plsc

_E = 8
_K = 2
_D = 1024
_F = 4096
_N = 4096
_BM = 256
_NBLK = 40
_PAD = _BM * _NBLK
_TB = 512
_NTB = _N // _TB

_NC = 2
_NS = 16
_NW = _NC * _NS
_TOK_W = _N // _NW
_CT = 32
_DNCH = _TOK_W // _CT
_CCT = 8
_CNCH = _TOK_W // _CCT


def _top2(probs, col):
    m1 = jnp.max(probs, axis=1, keepdims=True)
    i1 = jnp.min(jnp.where(probs == m1, col, _E), axis=1, keepdims=True)
    pm = jnp.where(col == i1, -1.0, probs)
    m2 = jnp.max(pm, axis=1, keepdims=True)
    i2 = jnp.min(jnp.where(pm == m2, col, _E), axis=1, keepdims=True)
    return m1, i1, m2, i2


def _router_body(x_ref, gw_ref, w_ref, pos_ref, ps_ref, be_ref, cnt_ref,
                 pr_ref):
    p = pl.program_id(0)
    blk = pl.program_id(1)
    col = lax.broadcasted_iota(jnp.int32, (_TB, _E), 1)

    @pl.when(p == 0)
    def _pass1_probs():
        x = x_ref[...]
        gw = gw_ref[...]
        logits = lax.dot_general(x, gw, (((1,), (1,)), ((), ())),
                                 preferred_element_type=jnp.float32)
        m = jnp.max(logits, axis=1, keepdims=True)
        ex = jnp.exp(logits - m)
        pr_ref[pl.ds(blk * _TB, _TB), :] = ex / jnp.sum(ex, axis=1,
                                                        keepdims=True)

    probs = pr_ref[pl.ds(blk * _TB, _TB), :]
    m1, i1, m2, i2 = _top2(probs, col)
    oh1 = jnp.where(col == i1, 1.0, 0.0)
    oh2 = jnp.where(col == i2, 1.0, 0.0)

    @pl.when(p == 0)
    def _pass1():
        @pl.when(blk == 0)
        def _():
            ps_ref[...] = jnp.zeros_like(ps_ref)

        ps_ref[...] += jnp.sum(probs, axis=0, keepdims=True)
        cnt_ref[blk, :] = jnp.sum(oh1 + oh2, axis=0)

    @pl.when(p == 1)
    def _pass2():
        s = m1 + m2 + 1e-9
        col32 = lax.broadcasted_iota(jnp.int32, (_TB, 2 * 16), 1)
        w_ref[...] = jnp.where(col32 < 16, m1 / s, m2 / s)
        cnt = cnt_ref[...]
        totals = jnp.sum(cnt, axis=0, keepdims=True)
        blocks_e = jnp.floor((totals + (_BM - 1)) * (1.0 / _BM))
        ecol = lax.broadcasted_iota(jnp.int32, (_E, _E), 1)
        erow = lax.broadcasted_iota(jnp.int32, (_E, _E), 0)
        tri_inc = jnp.where(erow <= ecol, 1.0, 0.0)
        cumb = lax.dot_general(blocks_e, tri_inc,
                               (((1,), (0,)), ((), ())),
                               preferred_element_type=jnp.float32)
        bstart = cumb - blocks_e
        brow = lax.broadcasted_iota(jnp.int32, (_NTB, _E), 0)
        bprefix = jnp.sum(jnp.where(brow < blk, cnt, 0.0), axis=0,
                          keepdims=True)
        trow = lax.broadcasted_iota(jnp.int32, (_TB, _TB), 0)
        tcol = lax.broadcasted_iota(jnp.int32, (_TB, _TB), 1)
        ltri = jnp.where(tcol < trow, 1.0, 0.0)
        pfx = lax.dot_general(ltri, oh1 + oh2, (((1,), (0,)), ((), ())),
                              preferred_element_type=jnp.float32)
        base = bstart * _BM + bprefix + pfx
        pos0 = jnp.sum(oh1 * base, axis=1, keepdims=True)
        pos1 = jnp.sum(oh2 * base, axis=1, keepdims=True)
        pos_ref[...] = jnp.where(col == 0, pos0,
                                 jnp.where(col == 1, pos1, 0.0)
                                 ).astype(jnp.int32)

        @pl.when(blk == 0)
        def _():
            bro = lax.broadcasted_iota(jnp.int32, (_NBLK + 1, _E), 0
                                       ).astype(jnp.float32)
            be = jnp.minimum(jnp.sum(jnp.where(cumb <= bro, 1.0, 0.0),
                                     axis=1, keepdims=True), _E - 1)
            lane = lax.broadcasted_iota(jnp.int32, (1, _E), 1)
            tot = jnp.sum(jnp.where(lane == _E - 1, cumb, 0.0), axis=1,
                          keepdims=True)
            rowi = lax.broadcasted_iota(jnp.int32, (_NBLK + 1, 1), 0)
            be_ref[...] = jnp.where(rowi < _NBLK, be, tot).astype(jnp.int32)


def _router(x2d, gate_w):
    return pl.pallas_call(
        _router_body,
        grid=(2, _NTB),
        in_specs=[
            pl.BlockSpec((_TB, _D), lambda p, b: (b * (1 - p), 0)),
            pl.BlockSpec((_E, _D), lambda p, b: (0, 0)),
        ],
        out_specs=[
            pl.BlockSpec((_TB, 32), lambda p, b: (b, 0)),
            pl.BlockSpec((_TB, _E), lambda p, b: (b, 0)),
            pl.BlockSpec((1, _E), lambda p, b: (0, 0)),
            pl.BlockSpec((_NBLK + 1, 1), lambda p, b: (0, 0)),
        ],
        out_shape=[
            jax.ShapeDtypeStruct((_N, 32), jnp.float32),
            jax.ShapeDtypeStruct((_N, _E), jnp.int32),
            jax.ShapeDtypeStruct((1, _E), jnp.float32),
            jax.ShapeDtypeStruct((_NBLK + 1, 1), jnp.int32),
        ],
        scratch_shapes=[pltpu.VMEM((_NTB, _E), jnp.float32),
                        pltpu.VMEM((_N, _E), jnp.float32)],
    )(x2d, gate_w)


def _ffn_body(be_ref, x_ref, w1_ref, b1_ref, w2_ref, b2_ref, out_ref,
              w1b, w2b):
    i = pl.program_id(1)
    j = pl.program_id(0)
    prev = be_ref[jnp.maximum(i - 1, 0)]
    changed = jnp.logical_or(i == 0, be_ref[i] != prev)

    @pl.when(jnp.logical_and(i < be_ref[_NBLK], changed))
    def _():
        w1b[...] = w1_ref[0].astype(jnp.bfloat16)
        w2b[...] = w2_ref[0].astype(jnp.bfloat16)

    @pl.when(i < be_ref[_NBLK])
    def _():
        x = x_ref[...].astype(jnp.bfloat16)
        h = lax.dot_general(x, w1b[...], (((1,), (1,)), ((), ())),
                            preferred_element_type=jnp.float32)
        h = h + b1_ref[0]
        h = 0.5 * h * (1.0 + lax.erf(h * (2.0 ** -0.5)))
        y = lax.dot_general(h.astype(jnp.bfloat16), w2b[...],
                            (((1,), (1,)), ((), ())),
                            preferred_element_type=jnp.float32)
        out_ref[0] = jnp.where(j == 1, y + b2_ref[0], y)


_FH = _F // 2


def _ffn(x_sorted, w1, b1, w2, b2, block_expert):
    grid_spec = pltpu.PrefetchScalarGridSpec(
        num_scalar_prefetch=1,
        grid=(2, _NBLK),
        in_specs=[
            pl.BlockSpec((_BM, _D), lambda j, i, be: (i, 0)),
            pl.BlockSpec((1, _FH, _D), lambda j, i, be: (be[i], j, 0)),
            pl.BlockSpec((1, 1, _FH), lambda j, i, be: (be[i], 0, j)),
            pl.BlockSpec((1, _D, _FH), lambda j, i, be: (be[i], 0, j)),
            pl.BlockSpec((1, 1, _D), lambda j, i, be: (be[i], 0, 0)),
        ],
        out_specs=pl.BlockSpec((1, _BM, _D), lambda j, i, be: (j, i, 0)),
        scratch_shapes=[pltpu.VMEM((_FH, _D), jnp.bfloat16),
                        pltpu.VMEM((_D, _FH), jnp.bfloat16)],
    )
    return pl.pallas_call(
        _ffn_body,
        grid_spec=grid_spec,
        out_shape=jax.ShapeDtypeStruct((2, _PAD, _D), jnp.float32),
    )(block_expert, x_sorted, w1, b1.reshape(_E, 1, _F), w2,
      b2.reshape(_E, 1, _D))


def _sc_mesh():
    return plsc.VectorSubcoreMesh(core_axis_name="c", subcore_axis_name="s",
                                  num_cores=_NC, num_subcores=_NS)


def _sc_dispatch_call(x2d, pidx):
    @functools.partial(
        pl.kernel,
        out_type=jax.ShapeDtypeStruct((_PAD, _D), jnp.float32),
        mesh=_sc_mesh(),
        scratch_types=[
            pltpu.VMEM((2 * _DNCH, _CT), jnp.int32),
            pltpu.VMEM((2, _CT, _D), jnp.float32),
            pltpu.SemaphoreType.DMA,
            pltpu.SemaphoreType.DMA,
        ],
    )
    def k(x_hbm, idx_hbm, out_hbm, idx_v, rows_v, lsem, ssem):
        wid = lax.axis_index("s") * _NC + lax.axis_index("c")
        tbase = wid * _TOK_W
        pltpu.sync_copy(idx_hbm.at[wid], idx_v)
        ld = [None] * _DNCH
        st = [[None, None] for _ in range(_DNCH)]
        ld[0] = pltpu.async_copy(
            x_hbm.at[pl.ds(tbase, _CT)], rows_v.at[0], lsem)
        for ch in range(_DNCH):
            b = ch & 1
            if ch + 1 < _DNCH:
                if ch >= 1:
                    st[ch - 1][0].wait()
                    st[ch - 1][1].wait()
                ld[ch + 1] = pltpu.async_copy(
                    x_hbm.at[pl.ds(tbase + (ch + 1) * _CT, _CT)],
                    rows_v.at[(ch + 1) & 1], lsem)
            ld[ch].wait()
            st[ch][0] = pltpu.async_copy(
                rows_v.at[b], out_hbm.at[idx_v.at[2 * ch]], ssem)
            st[ch][1] = pltpu.async_copy(
                rows_v.at[b], out_hbm.at[idx_v.at[2 * ch + 1]], ssem)
        for ch in (_DNCH - 2, _DNCH - 1):
            st[ch][0].wait()
            st[ch][1].wait()

    return k(x2d, pidx)


def _sc_combine_call(y_sorted, cidx, cw):
    @functools.partial(
        pl.kernel,
        out_type=jax.ShapeDtypeStruct((_N, _D), jnp.float32),
        mesh=_sc_mesh(),
        scratch_types=[
            pltpu.VMEM((_CNCH, 4 * _CCT), jnp.int32),
            pltpu.VMEM((_TOK_W, 32), jnp.float32),
            pltpu.VMEM((2, 4 * _CCT, _D), jnp.float32),
            pltpu.VMEM((2, _CCT, _D), jnp.float32),
            pltpu.SemaphoreType.DMA,
            pltpu.SemaphoreType.DMA,
        ],
    )
    def k(y_hbm, idx_hbm, w_hbm, out_hbm, idx_v, w_v, rows_v, out_v,
          gsem, ssem):
        wid = lax.axis_index("s") * _NC + lax.axis_index("c")
        base = wid * _TOK_W
        pltpu.sync_copy(idx_hbm.at[wid], idx_v)
        pltpu.sync_copy(w_hbm.at[wid], w_v)

        def chunk_compute(ch, b):
            def tok_body(i, _):
                w0 = w_v[ch * _CCT + i, pl.ds(0, 16)]
                w1 = w_v[ch * _CCT + i, pl.ds(16, 16)]
                for l in range(_D // 16):
                    sl = pl.ds(l * 16, 16)
                    out_v[b, i, sl] = (
                        w0 * (rows_v[b, 4 * i, sl] + rows_v[b, 4 * i + 1, sl])
                        + w1 * (rows_v[b, 4 * i + 2, sl]
                                + rows_v[b, 4 * i + 3, sl]))
                return 0

            lax.fori_loop(0, _CCT, tok_body, 0)

        def pair_body(it, _, first=False):
            c0 = 2 * it
            g0 = pltpu.async_copy(y_hbm.at[idx_v.at[c0]], rows_v.at[0], gsem)
            g1 = pltpu.async_copy(y_hbm.at[idx_v.at[c0 + 1]], rows_v.at[1],
                                  gsem)
            if not first:
                pltpu.make_async_copy(
                    out_v.at[0],
                    out_hbm.at[pl.ds(base + (c0 - 2) * _CCT, _CCT)],
                    ssem).wait()
            g0.wait()
            chunk_compute(c0, 0)
            pltpu.async_copy(
                out_v.at[0], out_hbm.at[pl.ds(base + c0 * _CCT, _CCT)], ssem)
            if not first:
                pltpu.make_async_copy(
                    out_v.at[1],
                    out_hbm.at[pl.ds(base + (c0 - 1) * _CCT, _CCT)],
                    ssem).wait()
            g1.wait()
            chunk_compute(c0 + 1, 1)
            pltpu.async_copy(
                out_v.at[1],
                out_hbm.at[pl.ds(base + (c0 + 1) * _CCT, _CCT)], ssem)
            return 0

        pair_body(0, 0, first=True)
        lax.fori_loop(1, _CNCH // 2, pair_body, 0)
        last = _CNCH - 2
        pltpu.make_async_copy(
            out_v.at[0], out_hbm.at[pl.ds(base + last * _CCT, _CCT)],
            ssem).wait()
        pltpu.make_async_copy(
            out_v.at[1], out_hbm.at[pl.ds(base + (last + 1) * _CCT, _CCT)],
            ssem).wait()

    return k(y_sorted, cidx, cw)


def kernel(x, gate_w, w1, b1, w2, b2):
    B, T, D = x.shape
    x2d = x.reshape(_N, _D)
    wts, posk, psum, bexp = _router(x2d, gate_w)

    avg = psum[0] / _N
    aux = jnp.mean((avg - 1.0 / _E) ** 2) * _E

    pos2 = posk[:, :_K]
    pidx = (pos2.reshape(_NW, _DNCH, _CT, _K)
            .transpose(0, 1, 3, 2).reshape(_NW, 2 * _DNCH, _CT))
    p0, p1 = pos2[:, 0], pos2[:, 1]
    cidx = (jnp.stack([p0, p0 + _PAD, p1, p1 + _PAD], axis=1)
            .reshape(_NW, _CNCH, 4 * _CCT))
    cw = wts.reshape(_NW, _TOK_W, 32)

    x_sorted = _sc_dispatch_call(x2d, pidx)
    y2 = _ffn(x_sorted, w1, b1, w2, b2, bexp.reshape(_NBLK + 1))
    out2d = _sc_combine_call(y2.reshape(2 * _PAD, _D), cidx, cw)
    return out2d.reshape(B, T, D), aux
